# baseline (speedup 1.0000x reference)
import functools

import jax
import jax.numpy as jnp
from jax import lax
from jax.experimental import pallas as pl
from jax.experimental.pallas import tpu as pltpu
from jax.experimental.pallas import tpu_sc as plsc

NFEAT = 128
H0, H1, H2 = 256, 128, 64


def _mlp_body(x_ref, w0t, b0r, w1t, b1r, w2t, b2r, w3t, b3r, w3row, w2f, w1f, w0f,
              etot_ref, ei_ref, de_ref):
    x = x_ref[0]
    f0 = jnp.dot(x, w0t[...], preferred_element_type=jnp.float32) + b0r[...]
    l0 = jax.nn.softplus(f0)
    d0 = jax.nn.sigmoid(f0)
    f1 = jnp.dot(l0, w1t[...], preferred_element_type=jnp.float32) + b1r[...]
    l1 = jax.nn.softplus(f1)
    d1 = jax.nn.sigmoid(f1)
    f2 = jnp.dot(l1, w2t[...], preferred_element_type=jnp.float32) + b2r[...]
    l2 = jax.nn.softplus(f2)
    d2 = jax.nn.sigmoid(f2)
    ei = jnp.dot(l2, w3t[...], preferred_element_type=jnp.float32) + b3r[...]
    etot_ref[...] = jnp.sum(ei).reshape(1, 1, 1)
    ei_ref[0] = ei
    g = d2 * w3row[...]
    g = jnp.dot(g, w2f[...], preferred_element_type=jnp.float32)
    g = d1 * g
    g = jnp.dot(g, w1f[...], preferred_element_type=jnp.float32)
    g = d0 * g
    de_ref[0] = jnp.dot(g, w0f[...], preferred_element_type=jnp.float32)


def _force_body(denb_ref, dft_ref, out_ref, *, nblk, knb):
    de3 = denb_ref[0].reshape(nblk, 1, knb, NFEAT)
    p = de3 * dft_ref[0]
    s = p.sum(axis=2).sum(axis=2)
    out_ref[0] = s


def _make_sc_gather(total_rows, feat, chunk):
    info = plsc.get_sparse_core_info()
    nw = info.num_cores * info.num_subcores
    per_w = total_rows // nw
    nchunk = per_w // chunk
    mesh = plsc.VectorSubcoreMesh(core_axis_name="c", subcore_axis_name="s")

    @functools.partial(
        pl.kernel, mesh=mesh,
        out_type=jax.ShapeDtypeStruct((total_rows, feat), jnp.float32),
        scratch_types=[
            pltpu.VMEM((chunk,), jnp.int32),
            pltpu.VMEM((chunk, feat), jnp.float32),
            pltpu.SemaphoreType.DMA,
        ],
    )
    def sc_gather(table_hbm, idx_hbm, out_hbm, idx_v, rows_v, sem):
        wid = lax.axis_index("s") * info.num_cores + lax.axis_index("c")
        base = wid * per_w
        for c in range(nchunk):
            off = base + c * chunk
            pltpu.sync_copy(idx_hbm.at[pl.ds(off, chunk)], idx_v)
            pltpu.async_copy(table_hbm.at[idx_v], rows_v, sem).wait()
            pltpu.sync_copy(rows_v, out_hbm.at[pl.ds(off, chunk)])

    return sc_gather


def kernel(image, dfeat, neighbor, Egroup_weight, divider,
           W0, b0, W1, b1, W2, b2, W3, b3):
    B, N, F = image.shape
    K = neighbor.shape[2]
    w0t, w1t, w2t, w3t = W0.T, W1.T, W2.T, W3.T
    b0r, b1r, b2r = b0[None, :], b1[None, :], b2[None, :]
    b3r = b3[None, :]
    dfeat_t = dfeat.transpose(0, 1, 4, 2, 3)
    gidx = (neighbor.astype(jnp.int32) - 1
            + (jnp.arange(B, dtype=jnp.int32) * N)[:, None, None]).reshape(-1)

    full = lambda s: pl.BlockSpec(s, lambda b: (0,) * len(s))
    etot, ei, de = pl.pallas_call(
        _mlp_body,
        grid=(B,),
        in_specs=[
            pl.BlockSpec((1, N, F), lambda b: (b, 0, 0)),
            full((F, H0)), full((1, H0)),
            full((H0, H1)), full((1, H1)),
            full((H1, H2)), full((1, H2)),
            full((H2, 1)), full((1, 1)),
            full((1, H2)), full((H2, H1)), full((H1, H0)), full((H0, F)),
        ],
        out_specs=[
            pl.BlockSpec((1, 1, 1), lambda b: (b, 0, 0)),
            pl.BlockSpec((1, N, 1), lambda b: (b, 0, 0)),
            pl.BlockSpec((1, N, F), lambda b: (b, 0, 0)),
        ],
        out_shape=[
            jax.ShapeDtypeStruct((B, 1, 1), jnp.float32),
            jax.ShapeDtypeStruct((B, N, 1), jnp.float32),
            jax.ShapeDtypeStruct((B, N, F), jnp.float32),
        ],
    )(image, w0t, b0r, w1t, b1r, w2t, b2r, w3t, b3r, W3, W2, W1, W0)

    de_nb = _make_sc_gather(B * N * K, F, 128)(de.reshape(B * N, F), gidx)
    de_nb = de_nb.reshape(B, N, K, F)

    NBLK = 128
    force = pl.pallas_call(
        functools.partial(_force_body, nblk=NBLK, knb=K),
        grid=(B, N // NBLK),
        in_specs=[
            pl.BlockSpec((1, NBLK, K, F), lambda b, i: (b, i, 0, 0)),
            pl.BlockSpec((1, NBLK, 3, K, F), lambda b, i: (b, i, 0, 0, 0)),
        ],
        out_specs=pl.BlockSpec((1, NBLK, 3), lambda b, i: (b, i, 0)),
        out_shape=jax.ShapeDtypeStruct((B, N, 3), jnp.float32),
    )(de_nb, dfeat_t)

    return (etot.reshape(B, 1), ei, force)


# speedup vs baseline: 2.6736x; 2.6736x over previous
import functools

import jax
import jax.numpy as jnp
from jax import lax
from jax.experimental import pallas as pl
from jax.experimental.pallas import tpu as pltpu

NFEAT = 128
H0, H1, H2 = 256, 128, 64


def _fused_body(nb_ref, image_ref, w0t, b0r, w1t, b1r, w2t, b2r, w3t, b3r,
                w3row, w2f, w1f, w0f, dft_ref,
                etot_ref, ei_ref, out_ref, de_scr, *, nblk, natom, knb):
    @pl.when(pl.program_id(1) == 0)
    def _mlp():
        x = image_ref[0]
        f0 = jnp.dot(x, w0t[...], preferred_element_type=jnp.float32) + b0r[...]
        l0 = jax.nn.softplus(f0)
        d0 = jax.nn.sigmoid(f0)
        f1 = jnp.dot(l0, w1t[...], preferred_element_type=jnp.float32) + b1r[...]
        l1 = jax.nn.softplus(f1)
        d1 = jax.nn.sigmoid(f1)
        f2 = jnp.dot(l1, w2t[...], preferred_element_type=jnp.float32) + b2r[...]
        l2 = jax.nn.softplus(f2)
        d2 = jax.nn.sigmoid(f2)
        ei = jnp.dot(l2, w3t[...], preferred_element_type=jnp.float32) + b3r[...]
        etot_ref[...] = jnp.sum(ei).reshape(1, 1, 1)
        ei_ref[0] = ei
        g = d2 * w3row[...]
        g = jnp.dot(g, w2f[...], preferred_element_type=jnp.float32)
        g = d1 * g
        g = jnp.dot(g, w1f[...], preferred_element_type=jnp.float32)
        g = d0 * g
        de = jnp.dot(g, w0f[...], preferred_element_type=jnp.float32)
        de_scr[...] = de.astype(jnp.bfloat16)

    rows = nblk * knb
    nb = nb_ref[0] - 1
    iota = lax.broadcasted_iota(jnp.int32, (nblk, knb, natom), 2)
    oh = (nb[:, :, None] == iota).astype(jnp.bfloat16)
    oh2 = oh.reshape(rows, natom)
    de_nb = jnp.dot(oh2, de_scr[...], preferred_element_type=jnp.float32)
    de3 = de_nb.reshape(nblk, 1, knb, NFEAT)
    p = de3 * dft_ref[0]
    s = p.sum(axis=2).sum(axis=2)
    out_ref[0] = s


def kernel(image, dfeat, neighbor, Egroup_weight, divider,
           W0, b0, W1, b1, W2, b2, W3, b3):
    B, N, F = image.shape
    K = neighbor.shape[2]
    w0t, w1t, w2t, w3t = W0.T, W1.T, W2.T, W3.T
    b0r, b1r, b2r = b0[None, :], b1[None, :], b2[None, :]
    b3r = b3[None, :]
    nb = neighbor.astype(jnp.int32)
    dfeat_t = dfeat.transpose(0, 1, 4, 2, 3)

    NBLK = 128
    full = lambda s: pl.BlockSpec(s, lambda b, i: (0,) * len(s))
    etot, ei, force = pl.pallas_call(
        functools.partial(_fused_body, nblk=NBLK, natom=N, knb=K),
        grid=(B, N // NBLK),
        in_specs=[
            pl.BlockSpec((1, NBLK, K), lambda b, i: (b, i, 0)),
            pl.BlockSpec((1, N, F), lambda b, i: (b, 0, 0)),
            full((F, H0)), full((1, H0)),
            full((H0, H1)), full((1, H1)),
            full((H1, H2)), full((1, H2)),
            full((H2, 1)), full((1, 1)),
            full((1, H2)), full((H2, H1)), full((H1, H0)), full((H0, F)),
            pl.BlockSpec((1, NBLK, 3, K, F), lambda b, i: (b, i, 0, 0, 0)),
        ],
        out_specs=[
            pl.BlockSpec((1, 1, 1), lambda b, i: (b, 0, 0)),
            pl.BlockSpec((1, N, 1), lambda b, i: (b, 0, 0)),
            pl.BlockSpec((1, NBLK, 3), lambda b, i: (b, i, 0)),
        ],
        out_shape=[
            jax.ShapeDtypeStruct((B, 1, 1), jnp.float32),
            jax.ShapeDtypeStruct((B, N, 1), jnp.float32),
            jax.ShapeDtypeStruct((B, N, 3), jnp.float32),
        ],
        scratch_shapes=[pltpu.VMEM((N, F), jnp.bfloat16)],
    )(nb, image, w0t, b0r, w1t, b1r, w2t, b2r, w3t, b3r, W3, W2, W1, W0, dfeat_t)

    return (etot.reshape(B, 1), ei, force)
